# baseline (speedup 1.0000x reference)
import functools

import jax
import jax.numpy as jnp
from jax import lax
from jax.experimental import pallas as pl
from jax.experimental.pallas import tpu as pltpu
from jax.experimental.pallas import tpu_sc as plsc

N_NODES = 50000
E_REAL = 800000
NW = 32
EW = E_REAL // NW
CB = 1000
CBP = 1008
NCHUNK = EW // CB
NACC = 50048
RT = NACC // 16
TB = 400
EB = TB * 8
NT = 2000

_MESH = plsc.VectorSubcoreMesh(core_axis_name="c", subcore_axis_name="s")
_SC_PARAMS = pltpu.CompilerParams(use_tc_tiling_on_sc=False,
                                  needs_layout_passes=False)


def _leaky(x):
    return jnp.where(x >= 0, x, 0.01 * x)


def _splat(c):
    return jnp.full((16,), c, jnp.int32)


def _t0_body(h8_ref, w4_ref, gh_ref):
    gh_ref[...] = jnp.dot(h8_ref[...], w4_ref[...],
                          preferred_element_type=jnp.float32)


def _t1_body(hsd_ref, w1_ref, bias_ref, b2_ref, w2_ref, exrm_ref):
    f_tmp = (jnp.dot(hsd_ref[...].astype(jnp.bfloat16), w1_ref[...],
                     preferred_element_type=jnp.float32)
             + bias_ref[...])
    f_out = _leaky(f_tmp)
    red = jnp.dot(f_out.astype(jnp.bfloat16), w2_ref[...],
                  preferred_element_type=jnp.float32)
    red = red + b2_ref[...]
    col = lax.broadcasted_iota(jnp.int32, (TB, 128), 1)
    exrm_ref[...] = jnp.where(col % 16 < 4, jnp.exp(red), red)


def _t2_body(dp_ref, out_ref):
    out_ref[...] = 1.0 / (dp_ref[0] + dp_ref[1] + 1e-16)


def _t3_body(pp_ref, out_ref):
    out_ref[...] = pp_ref[0] + pp_ref[1]


def _s1_body(h16s, h16d, srcg, dstg, distf, istf, hsd_out,
             idxs_v, idxd_v, hsb, dch, ich, sem):
    wid = lax.axis_index("c") * 16 + lax.axis_index("s")
    iota = lax.iota(jnp.int32, 16)

    def chunk(jc, carry):
        base = wid * EW + jc * CB
        pltpu.sync_copy(srcg.at[pl.ds(base, CB)], idxs_v)
        pltpu.sync_copy(dstg.at[pl.ds(base, CB)], idxd_v)
        pltpu.sync_copy(distf.at[pl.ds(base, CB)], dch.at[pl.ds(0, CB)])
        pltpu.sync_copy(istf.at[pl.ds(base, CB)], ich.at[pl.ds(0, CB)])
        pltpu.async_copy(h16s.at[idxs_v], hsb.at[pl.ds(0, CB)], sem).wait()
        pltpu.async_copy(h16d.at[idxd_v], hsb.at[pl.ds(0, CB)], sem,
                         add=True).wait()

        def grp(g, c2):
            e16 = g * 16 + iota
            dv = plsc.load_gather(dch, [e16])
            iv = plsc.load_gather(ich, [e16]).astype(jnp.float32)
            plsc.store_scatter(hsb, [e16, _splat(3)], dv)
            plsc.store_scatter(hsb, [e16, _splat(11)], iv)
            return c2

        lax.fori_loop(0, CBP // 16, grp, 0)
        pltpu.sync_copy(hsb.at[pl.ds(0, CB)], hsd_out.at[pl.ds(base, CB)])
        return carry

    lax.fori_loop(0, NCHUNK, chunk, 0)


def _s2_body(dstg, exrm, zeros16, dp_out, idxd_v, exb, shared):
    cid = lax.axis_index("c")
    sid = lax.axis_index("s")
    wid = cid * 16 + sid
    pltpu.sync_copy(zeros16.at[pl.ds(sid * RT, RT)],
                    shared.at[pl.ds(sid * RT, RT)])
    plsc.subcore_barrier()

    def chunk(jc, carry):
        base = wid * EW + jc * CB
        pltpu.sync_copy(dstg.at[pl.ds(base, CB)], idxd_v)
        pltpu.sync_copy(exrm.at[pl.ds(base, CB)], exb)
        pltpu.sync_copy(exb, shared.at[idxd_v], add=True)
        return carry

    lax.fori_loop(0, NCHUNK, chunk, 0)
    plsc.subcore_barrier()
    pltpu.sync_copy(shared.at[pl.ds(sid * RT, RT)],
                    dp_out.at[cid, pl.ds(sid * RT, RT)])


def _s3_body(srcg, dstg, exrm, dr_hbm, gh_hbm, zeros8, pqp_out,
             idxs_v, idxd_v, exb, drows, ghrows, pqb, shared16, sem):
    cid = lax.axis_index("c")
    sid = lax.axis_index("s")
    wid = cid * 16 + sid
    pltpu.sync_copy(zeros8.at[pl.ds(sid * RT, RT)],
                    shared16.at[pl.ds(sid * RT, RT)])
    plsc.subcore_barrier()
    iota = lax.iota(jnp.int32, 16)

    def chunk(jc, carry):
        base = wid * EW + jc * CB
        pltpu.sync_copy(srcg.at[pl.ds(base, CB)], idxs_v)
        pltpu.sync_copy(dstg.at[pl.ds(base, CB)], idxd_v)
        pltpu.sync_copy(exrm.at[pl.ds(base, CB)], exb.at[pl.ds(0, CB)])
        cp1 = pltpu.async_copy(dr_hbm.at[idxd_v], drows.at[pl.ds(0, CB)], sem)
        cp2 = pltpu.async_copy(gh_hbm.at[idxs_v], ghrows.at[pl.ds(0, CB)], sem)
        cp1.wait()
        cp2.wait()

        def grp(g, c2):
            e16 = g * 16 + iota
            a = [plsc.load_gather(exb, [e16, _splat(h)])
                 * plsc.load_gather(drows, [e16, _splat(h)])
                 for h in range(4)]
            for k in range(4):
                accg = a[0] * plsc.load_gather(ghrows, [e16, _splat(k)])
                acch = a[0] * plsc.load_gather(ghrows, [e16, _splat(16 + k)])
                for h in range(1, 4):
                    accg += a[h] * plsc.load_gather(
                        ghrows, [e16, _splat(h * 4 + k)])
                    acch += a[h] * plsc.load_gather(
                        ghrows, [e16, _splat(16 + h * 4 + k)])
                plsc.store_scatter(pqb, [e16, _splat(k)], accg)
                plsc.store_scatter(pqb, [e16, _splat(4 + k)], acch)
            return c2

        lax.fori_loop(0, CBP // 16, grp, 0)
        pltpu.sync_copy(pqb.at[pl.ds(0, CB)], shared16.at[idxd_v], add=True)
        return carry

    lax.fori_loop(0, NCHUNK, chunk, 0)
    plsc.subcore_barrier()
    pltpu.sync_copy(shared16.at[pl.ds(sid * RT, RT)],
                    pqp_out.at[cid, pl.ds(sid * RT, RT)])


def _s4_body(srcg, dstg, exrm, pq_hbm, out_hbm,
             idxs_v, idxd_v, exb, pqs, pqd, outb, sem):
    wid = lax.axis_index("c") * 16 + lax.axis_index("s")
    iota = lax.iota(jnp.int32, 16)

    def chunk(jc, carry):
        base = wid * EW + jc * CB
        pltpu.sync_copy(srcg.at[pl.ds(base, CB)], idxs_v)
        pltpu.sync_copy(dstg.at[pl.ds(base, CB)], idxd_v)
        pltpu.sync_copy(exrm.at[pl.ds(base, CB)], exb.at[pl.ds(0, CB)])
        cp1 = pltpu.async_copy(pq_hbm.at[idxs_v], pqs.at[pl.ds(0, CB)], sem)
        cp2 = pltpu.async_copy(pq_hbm.at[idxd_v], pqd.at[pl.ds(0, CB)], sem)
        cp1.wait()
        cp2.wait()

        def grp(g, c2):
            e16 = g * 16 + iota
            acc = jnp.zeros((16,), jnp.float32)
            for k in range(4):
                x = (plsc.load_gather(pqs, [e16, _splat(k)])
                     + plsc.load_gather(pqd, [e16, _splat(4 + k)])
                     + plsc.load_gather(exb, [e16, _splat(4 + k)]))
                acc += jnp.maximum(x, 0.0) + 0.01 * jnp.minimum(x, 0.0)
            plsc.store_scatter(outb, [e16], acc)
            return c2

        lax.fori_loop(0, CBP // 16, grp, 0)
        pltpu.sync_copy(outb.at[pl.ds(0, CB)], out_hbm.at[pl.ds(base, CB)])
        return carry

    lax.fori_loop(0, NCHUNK, chunk, 0)


def kernel(node_feature, edge_index, distance, is_same_team, Wd, bd, emb,
           W_ni1, W_fij1, W_nj1, b_e1, attn1, W_node1,
           W_ni2, W_fij2, W_nj2, b_e2, attn2, W_node2):
    E = edge_index.shape[1]
    N = node_feature.shape[0]
    f32 = jnp.float32

    Wf1a = W_fij1[:32]
    Wf1b = W_fij1[32:]
    v1 = (Wd @ Wf1a)[0]
    T = emb @ Wf1b
    bias1 = bd @ Wf1a + b_e1 + T[0]
    dT = T[1] - T[0]
    z4 = jnp.zeros((4, 256), f32)
    Wcat = jnp.concatenate([W_ni1, v1[None], z4, W_nj1, dT[None], z4],
                           axis=0)
    A4 = attn1[0]
    Wred1 = (jnp.eye(4, dtype=f32)[:, None, :] * A4[:, :, None]).reshape(256, 4)
    Wtile = jnp.concatenate([W_fij2] * 4, axis=0) / 16.0
    Wred = jnp.concatenate([Wred1, Wtile, jnp.zeros((256, 8), f32)],
                           axis=1)
    b2row = jnp.concatenate([jnp.zeros((4,), f32), b_e2 / 4.0,
                             jnp.zeros((8,), f32)])
    WG = jnp.einsum('chd,dk->chk', W_node1.reshape(3, 4, 64), W_ni2) / 16.0
    WH = jnp.einsum('chd,dk->chk', W_node1.reshape(3, 4, 64), W_nj2) / 16.0
    W4 = jnp.concatenate([WG.reshape(3, 16), WH.reshape(3, 16)],
                         axis=1)
    W4 = jnp.concatenate([W4, jnp.zeros((5, 32), f32)], axis=0)

    h8 = jnp.pad(node_feature, ((0, 0), (0, 5)))
    h16s = jnp.pad(node_feature, ((0, 0), (0, 13)))
    h16d = jnp.pad(node_feature, ((0, 0), (8, 5)))
    srcg = edge_index[0]
    dstg = edge_index[1]
    distf = distance.reshape(E)
    istf = is_same_team.reshape(E)
    zeros16 = jnp.zeros((NACC, 16), f32)

    gh = pl.pallas_call(
        _t0_body,
        grid=(N // NT,),
        in_specs=[pl.BlockSpec((NT, 8), lambda i: (i, 0)),
                  pl.BlockSpec((8, 32), lambda i: (0, 0))],
        out_specs=pl.BlockSpec((NT, 32), lambda i: (i, 0)),
        out_shape=jax.ShapeDtypeStruct((N, 32), f32),
    )(h8, W4)

    s1 = pl.kernel(
        _s1_body,
        out_type=jax.ShapeDtypeStruct((E, 16), f32),
        mesh=_MESH,
        compiler_params=_SC_PARAMS,
        scratch_types=[
            pltpu.VMEM((CB,), jnp.int32),
            pltpu.VMEM((CB,), jnp.int32),
            pltpu.VMEM((CBP, 16), f32),
            pltpu.VMEM((CBP,), f32),
            pltpu.VMEM((CBP,), jnp.int32),
            pltpu.SemaphoreType.DMA,
        ],
    )
    hsd = s1(h16s, h16d, srcg, dstg, distf, istf)

    eye8 = jnp.eye(8, dtype=f32)
    W1bd = jnp.kron(eye8, Wcat).astype(jnp.bfloat16)
    W2bd = jnp.kron(eye8, Wred).astype(jnp.bfloat16)
    bias_t = jnp.tile(bias1, 8)[None]
    b2_t = jnp.tile(b2row, 8)[None]
    exrm_w = pl.pallas_call(
        _t1_body,
        grid=(E // EB,),
        in_specs=[
            pl.BlockSpec((TB, 128), lambda i: (i, 0)),
            pl.BlockSpec((128, 2048), lambda i: (0, 0)),
            pl.BlockSpec((1, 2048), lambda i: (0, 0)),
            pl.BlockSpec((1, 128), lambda i: (0, 0)),
            pl.BlockSpec((2048, 128), lambda i: (0, 0)),
        ],
        out_specs=pl.BlockSpec((TB, 128), lambda i: (i, 0)),
        out_shape=jax.ShapeDtypeStruct((E // 8, 128), f32),
    )(hsd.reshape(E // 8, 128), W1bd, bias_t, b2_t, W2bd)
    exrm = exrm_w.reshape(E, 16)

    s2 = pl.kernel(
        _s2_body,
        out_type=jax.ShapeDtypeStruct((2, NACC, 16), f32),
        mesh=_MESH,
        compiler_params=_SC_PARAMS,
        scratch_types=[
            pltpu.VMEM((CB,), jnp.int32),
            pltpu.VMEM((CB, 16), f32),
            pltpu.VMEM_SHARED((NACC, 16), f32),
        ],
    )
    dp = s2(dstg, exrm, zeros16)

    r16 = NACC * 16 // 512
    dr = pl.pallas_call(
        _t2_body,
        in_specs=[pl.BlockSpec((2, r16, 512), lambda: (0, 0, 0))],
        out_specs=pl.BlockSpec((r16, 512), lambda: (0, 0)),
        out_shape=jax.ShapeDtypeStruct((r16, 512), f32),
    )(dp.reshape(2, r16, 512)).reshape(NACC, 16)

    zeros8 = jnp.zeros((NACC, 8), f32)
    s3 = pl.kernel(
        _s3_body,
        out_type=jax.ShapeDtypeStruct((2, NACC, 8), f32),
        mesh=_MESH,
        compiler_params=_SC_PARAMS,
        scratch_types=[
            pltpu.VMEM((CB,), jnp.int32),
            pltpu.VMEM((CB,), jnp.int32),
            pltpu.VMEM((CBP, 16), f32),
            pltpu.VMEM((CBP, 16), f32),
            pltpu.VMEM((CBP, 32), f32),
            pltpu.VMEM((CBP, 8), f32),
            pltpu.VMEM_SHARED((NACC, 8), f32),
            pltpu.SemaphoreType.DMA,
        ],
    )
    pqp = s3(srcg, dstg, exrm, dr, gh, zeros8)

    r8 = NACC * 8 // 512
    pq = pl.pallas_call(
        _t3_body,
        in_specs=[pl.BlockSpec((2, r8, 512), lambda: (0, 0, 0))],
        out_specs=pl.BlockSpec((r8, 512), lambda: (0, 0)),
        out_shape=jax.ShapeDtypeStruct((r8, 512), f32),
    )(pqp.reshape(2, r8, 512)).reshape(NACC, 8)

    s4 = pl.kernel(
        _s4_body,
        out_type=jax.ShapeDtypeStruct((E,), f32),
        mesh=_MESH,
        compiler_params=_SC_PARAMS,
        scratch_types=[
            pltpu.VMEM((CB,), jnp.int32),
            pltpu.VMEM((CB,), jnp.int32),
            pltpu.VMEM((CBP, 16), f32),
            pltpu.VMEM((CBP, 8), f32),
            pltpu.VMEM((CBP, 8), f32),
            pltpu.VMEM((CBP,), f32),
            pltpu.SemaphoreType.DMA,
        ],
    )
    out = s4(srcg, dstg, exrm, pq)
    return out[:, None]


# speedup vs baseline: 1.0046x; 1.0046x over previous
import functools

import jax
import jax.numpy as jnp
from jax import lax
from jax.experimental import pallas as pl
from jax.experimental.pallas import tpu as pltpu
from jax.experimental.pallas import tpu_sc as plsc

N_NODES = 50000
E_REAL = 800000
NW = 32
EW = E_REAL // NW
CB = 1000
CBP = 1008
NCHUNK = EW // CB
NACC = 50048
RT = NACC // 16
TB = 400
EB = TB * 8
NT = 2000

_MESH = plsc.VectorSubcoreMesh(core_axis_name="c", subcore_axis_name="s")
_SC_PARAMS = pltpu.CompilerParams(use_tc_tiling_on_sc=False,
                                  needs_layout_passes=False)


def _leaky(x):
    return jnp.where(x >= 0, x, 0.01 * x)


def _splat(c):
    return jnp.full((16,), c, jnp.int32)


def _t0_body(h8_ref, w4_ref, gh_ref):
    gh_ref[...] = jnp.dot(h8_ref[...], w4_ref[...],
                          preferred_element_type=jnp.float32)


def _t1_body(hsd_ref, w1_ref, bias_ref, b2_ref, w2_ref, exrm_ref):
    f_tmp = (jnp.dot(hsd_ref[...], w1_ref[...],
                     preferred_element_type=jnp.float32).astype(jnp.bfloat16)
             + bias_ref[...])
    f_out = _leaky(f_tmp)
    red = jnp.dot(f_out, w2_ref[...], preferred_element_type=jnp.float32)
    red = red + b2_ref[...]
    col = lax.broadcasted_iota(jnp.int32, (TB, 128), 1)
    exrm_ref[...] = jnp.where(col % 16 < 4, jnp.exp(red), red)


def _t2_body(dp_ref, out_ref):
    out_ref[...] = 1.0 / (dp_ref[0] + dp_ref[1] + 1e-16)


def _t3_body(pp_ref, out_ref):
    out_ref[...] = pp_ref[0] + pp_ref[1]


def _s1_body(h16s, h16d, srcg, dstg, distf, istf, hsd_out,
             idxs_v, idxd_v, hsb, dch, ich, sem):
    wid = lax.axis_index("c") * 16 + lax.axis_index("s")
    iota = lax.iota(jnp.int32, 16)

    def chunk(jc, carry):
        base = wid * EW + jc * CB
        pltpu.sync_copy(srcg.at[pl.ds(base, CB)], idxs_v)
        pltpu.sync_copy(dstg.at[pl.ds(base, CB)], idxd_v)
        pltpu.sync_copy(distf.at[pl.ds(base, CB)], dch.at[pl.ds(0, CB)])
        pltpu.sync_copy(istf.at[pl.ds(base, CB)], ich.at[pl.ds(0, CB)])
        pltpu.async_copy(h16s.at[idxs_v], hsb.at[pl.ds(0, CB)], sem).wait()
        pltpu.async_copy(h16d.at[idxd_v], hsb.at[pl.ds(0, CB)], sem,
                         add=True).wait()

        def grp(g, c2):
            e16 = g * 16 + iota
            dv = plsc.load_gather(dch, [e16])
            iv = plsc.load_gather(ich, [e16]).astype(jnp.float32)
            plsc.store_scatter(hsb, [e16, _splat(3)], dv)
            plsc.store_scatter(hsb, [e16, _splat(11)], iv)
            return c2

        lax.fori_loop(0, CBP // 16, grp, 0)
        pltpu.sync_copy(hsb.at[pl.ds(0, CB)], hsd_out.at[pl.ds(base, CB)])
        return carry

    lax.fori_loop(0, NCHUNK, chunk, 0)


def _s2_body(dstg, exrm, zeros16, dp_out, idxd_v, exb, shared):
    cid = lax.axis_index("c")
    sid = lax.axis_index("s")
    wid = cid * 16 + sid
    pltpu.sync_copy(zeros16.at[pl.ds(sid * RT, RT)],
                    shared.at[pl.ds(sid * RT, RT)])
    plsc.subcore_barrier()

    def chunk(jc, carry):
        base = wid * EW + jc * CB
        pltpu.sync_copy(dstg.at[pl.ds(base, CB)], idxd_v)
        pltpu.sync_copy(exrm.at[pl.ds(base, CB)], exb)
        pltpu.sync_copy(exb, shared.at[idxd_v], add=True)
        return carry

    lax.fori_loop(0, NCHUNK, chunk, 0)
    plsc.subcore_barrier()
    pltpu.sync_copy(shared.at[pl.ds(sid * RT, RT)],
                    dp_out.at[cid, pl.ds(sid * RT, RT)])


def _s3_body(srcg, dstg, exrm, dr_hbm, gh_hbm, zeros8, pqp_out,
             idxs_v, idxd_v, exb, drows, ghrows, pqb, shared16, sem):
    cid = lax.axis_index("c")
    sid = lax.axis_index("s")
    wid = cid * 16 + sid
    pltpu.sync_copy(zeros8.at[pl.ds(sid * RT, RT)],
                    shared16.at[pl.ds(sid * RT, RT)])
    plsc.subcore_barrier()
    iota = lax.iota(jnp.int32, 16)

    def chunk(jc, carry):
        base = wid * EW + jc * CB
        pltpu.sync_copy(srcg.at[pl.ds(base, CB)], idxs_v)
        pltpu.sync_copy(dstg.at[pl.ds(base, CB)], idxd_v)
        pltpu.sync_copy(exrm.at[pl.ds(base, CB)], exb.at[pl.ds(0, CB)])
        cp1 = pltpu.async_copy(dr_hbm.at[idxd_v], drows.at[pl.ds(0, CB)], sem)
        cp2 = pltpu.async_copy(gh_hbm.at[idxs_v], ghrows.at[pl.ds(0, CB)], sem)
        cp1.wait()
        cp2.wait()

        def grp(g, c2):
            e16 = g * 16 + iota
            a = [plsc.load_gather(exb, [e16, _splat(h)])
                 * plsc.load_gather(drows, [e16, _splat(h)])
                 for h in range(4)]
            for k in range(4):
                accg = a[0] * plsc.load_gather(ghrows, [e16, _splat(k)])
                acch = a[0] * plsc.load_gather(ghrows, [e16, _splat(16 + k)])
                for h in range(1, 4):
                    accg += a[h] * plsc.load_gather(
                        ghrows, [e16, _splat(h * 4 + k)])
                    acch += a[h] * plsc.load_gather(
                        ghrows, [e16, _splat(16 + h * 4 + k)])
                plsc.store_scatter(pqb, [e16, _splat(k)], accg)
                plsc.store_scatter(pqb, [e16, _splat(4 + k)], acch)
            return c2

        lax.fori_loop(0, CBP // 16, grp, 0)
        pltpu.sync_copy(pqb.at[pl.ds(0, CB)], shared16.at[idxd_v], add=True)
        return carry

    lax.fori_loop(0, NCHUNK, chunk, 0)
    plsc.subcore_barrier()
    pltpu.sync_copy(shared16.at[pl.ds(sid * RT, RT)],
                    pqp_out.at[cid, pl.ds(sid * RT, RT)])


def _s4_body(srcg, dstg, exrm, pq_hbm, out_hbm,
             idxs_v, idxd_v, exb, pqs, pqd, outb, sem):
    wid = lax.axis_index("c") * 16 + lax.axis_index("s")
    iota = lax.iota(jnp.int32, 16)

    def chunk(jc, carry):
        base = wid * EW + jc * CB
        pltpu.sync_copy(srcg.at[pl.ds(base, CB)], idxs_v)
        pltpu.sync_copy(dstg.at[pl.ds(base, CB)], idxd_v)
        pltpu.sync_copy(exrm.at[pl.ds(base, CB)], exb.at[pl.ds(0, CB)])
        cp1 = pltpu.async_copy(pq_hbm.at[idxs_v], pqs.at[pl.ds(0, CB)], sem)
        cp2 = pltpu.async_copy(pq_hbm.at[idxd_v], pqd.at[pl.ds(0, CB)], sem)
        cp1.wait()
        cp2.wait()

        def grp(g, c2):
            e16 = g * 16 + iota
            acc = jnp.zeros((16,), jnp.float32)
            for k in range(4):
                x = (plsc.load_gather(pqs, [e16, _splat(k)])
                     + plsc.load_gather(pqd, [e16, _splat(4 + k)])
                     + plsc.load_gather(exb, [e16, _splat(4 + k)]))
                acc += jnp.maximum(x, 0.0) + 0.01 * jnp.minimum(x, 0.0)
            plsc.store_scatter(outb, [e16], acc)
            return c2

        lax.fori_loop(0, CBP // 16, grp, 0)
        pltpu.sync_copy(outb.at[pl.ds(0, CB)], out_hbm.at[pl.ds(base, CB)])
        return carry

    lax.fori_loop(0, NCHUNK, chunk, 0)


def kernel(node_feature, edge_index, distance, is_same_team, Wd, bd, emb,
           W_ni1, W_fij1, W_nj1, b_e1, attn1, W_node1,
           W_ni2, W_fij2, W_nj2, b_e2, attn2, W_node2):
    E = edge_index.shape[1]
    N = node_feature.shape[0]
    f32 = jnp.float32

    Wf1a = W_fij1[:32]
    Wf1b = W_fij1[32:]
    v1 = (Wd @ Wf1a)[0]
    T = emb @ Wf1b
    bias1 = bd @ Wf1a + b_e1 + T[0]
    dT = T[1] - T[0]
    z4 = jnp.zeros((4, 256), f32)
    Wcat = jnp.concatenate([W_ni1, v1[None], z4, W_nj1, dT[None], z4],
                           axis=0)
    A4 = attn1[0]
    Wred1 = (jnp.eye(4, dtype=f32)[:, None, :] * A4[:, :, None]).reshape(256, 4)
    Wtile = jnp.concatenate([W_fij2] * 4, axis=0) / 16.0
    Wred = jnp.concatenate([Wred1, Wtile, jnp.zeros((256, 8), f32)],
                           axis=1)
    b2row = jnp.concatenate([jnp.zeros((4,), f32), b_e2 / 4.0,
                             jnp.zeros((8,), f32)])
    WG = jnp.einsum('chd,dk->chk', W_node1.reshape(3, 4, 64), W_ni2) / 16.0
    WH = jnp.einsum('chd,dk->chk', W_node1.reshape(3, 4, 64), W_nj2) / 16.0
    W4 = jnp.concatenate([WG.reshape(3, 16), WH.reshape(3, 16)],
                         axis=1)
    W4 = jnp.concatenate([W4, jnp.zeros((5, 32), f32)], axis=0)

    h8 = jnp.pad(node_feature, ((0, 0), (0, 5)))
    h16s = jnp.pad(node_feature, ((0, 0), (0, 13)))
    h16d = jnp.pad(node_feature, ((0, 0), (8, 5)))
    srcg = edge_index[0]
    dstg = edge_index[1]
    distf = distance.reshape(E)
    istf = is_same_team.reshape(E)
    zeros16 = jnp.zeros((NACC, 16), f32)

    gh = pl.pallas_call(
        _t0_body,
        grid=(N // NT,),
        in_specs=[pl.BlockSpec((NT, 8), lambda i: (i, 0)),
                  pl.BlockSpec((8, 32), lambda i: (0, 0))],
        out_specs=pl.BlockSpec((NT, 32), lambda i: (i, 0)),
        out_shape=jax.ShapeDtypeStruct((N, 32), f32),
    )(h8, W4)

    s1 = pl.kernel(
        _s1_body,
        out_type=jax.ShapeDtypeStruct((E, 16), f32),
        mesh=_MESH,
        compiler_params=_SC_PARAMS,
        scratch_types=[
            pltpu.VMEM((CB,), jnp.int32),
            pltpu.VMEM((CB,), jnp.int32),
            pltpu.VMEM((CBP, 16), f32),
            pltpu.VMEM((CBP,), f32),
            pltpu.VMEM((CBP,), jnp.int32),
            pltpu.SemaphoreType.DMA,
        ],
    )
    hsd = s1(h16s, h16d, srcg, dstg, distf, istf)

    eye8 = jnp.eye(8, dtype=f32)
    W1bd = jnp.kron(eye8, Wcat)
    W2bd = jnp.kron(eye8, Wred).astype(jnp.bfloat16)
    bias_t = jnp.tile(bias1, 8)[None].astype(jnp.bfloat16)
    b2_t = jnp.tile(b2row, 8)[None]
    exrm_w = pl.pallas_call(
        _t1_body,
        grid=(E // EB,),
        in_specs=[
            pl.BlockSpec((TB, 128), lambda i: (i, 0)),
            pl.BlockSpec((128, 2048), lambda i: (0, 0)),
            pl.BlockSpec((1, 2048), lambda i: (0, 0)),
            pl.BlockSpec((1, 128), lambda i: (0, 0)),
            pl.BlockSpec((2048, 128), lambda i: (0, 0)),
        ],
        out_specs=pl.BlockSpec((TB, 128), lambda i: (i, 0)),
        out_shape=jax.ShapeDtypeStruct((E // 8, 128), f32),
    )(hsd.reshape(E // 8, 128), W1bd, bias_t, b2_t, W2bd)
    exrm = exrm_w.reshape(E, 16)

    s2 = pl.kernel(
        _s2_body,
        out_type=jax.ShapeDtypeStruct((2, NACC, 16), f32),
        mesh=_MESH,
        compiler_params=_SC_PARAMS,
        scratch_types=[
            pltpu.VMEM((CB,), jnp.int32),
            pltpu.VMEM((CB, 16), f32),
            pltpu.VMEM_SHARED((NACC, 16), f32),
        ],
    )
    dp = s2(dstg, exrm, zeros16)

    r16 = NACC * 16 // 512
    dr = pl.pallas_call(
        _t2_body,
        in_specs=[pl.BlockSpec((2, r16, 512), lambda: (0, 0, 0))],
        out_specs=pl.BlockSpec((r16, 512), lambda: (0, 0)),
        out_shape=jax.ShapeDtypeStruct((r16, 512), f32),
    )(dp.reshape(2, r16, 512)).reshape(NACC, 16)

    zeros8 = jnp.zeros((NACC, 8), f32)
    s3 = pl.kernel(
        _s3_body,
        out_type=jax.ShapeDtypeStruct((2, NACC, 8), f32),
        mesh=_MESH,
        compiler_params=_SC_PARAMS,
        scratch_types=[
            pltpu.VMEM((CB,), jnp.int32),
            pltpu.VMEM((CB,), jnp.int32),
            pltpu.VMEM((CBP, 16), f32),
            pltpu.VMEM((CBP, 16), f32),
            pltpu.VMEM((CBP, 32), f32),
            pltpu.VMEM((CBP, 8), f32),
            pltpu.VMEM_SHARED((NACC, 8), f32),
            pltpu.SemaphoreType.DMA,
        ],
    )
    pqp = s3(srcg, dstg, exrm, dr, gh, zeros8)

    r8 = NACC * 8 // 512
    pq = pl.pallas_call(
        _t3_body,
        in_specs=[pl.BlockSpec((2, r8, 512), lambda: (0, 0, 0))],
        out_specs=pl.BlockSpec((r8, 512), lambda: (0, 0)),
        out_shape=jax.ShapeDtypeStruct((r8, 512), f32),
    )(pqp.reshape(2, r8, 512)).reshape(NACC, 8)

    s4 = pl.kernel(
        _s4_body,
        out_type=jax.ShapeDtypeStruct((E,), f32),
        mesh=_MESH,
        compiler_params=_SC_PARAMS,
        scratch_types=[
            pltpu.VMEM((CB,), jnp.int32),
            pltpu.VMEM((CB,), jnp.int32),
            pltpu.VMEM((CBP, 16), f32),
            pltpu.VMEM((CBP, 8), f32),
            pltpu.VMEM((CBP, 8), f32),
            pltpu.VMEM((CBP,), f32),
            pltpu.SemaphoreType.DMA,
        ],
    )
    out = s4(srcg, dstg, exrm, pq)
    return out[:, None]
